# baseline (speedup 1.0000x reference)
import functools

import jax
import jax.numpy as jnp
from jax import lax
from jax.experimental import pallas as pl
from jax.experimental.pallas import tpu as pltpu
from jax.experimental.pallas import tpu_sc as plsc

H = 128
HH = H // 2
BOND = 16
ATOM = 128
N = 10000
E = 320000
M = 256
FFN = 300
NC, NS = 2, 16
E_S = E // NS
CH = 125
NCHUNK = E_S // CH
NBUF = 4
WST = 624
WCH = 104
NWO = WST // WCH
TAIL = N - NS * WST
BN = 1000


def _seg_body(with_bonds, *refs):
    if with_bonds:
        (h_hbm, eidx_hbm, fb_hbm, zrow_hbm, zbond_hbm,
         out_hbm, bout_hbm,
         idx_v, rows_v, fb_v, wrow_v, wbond_v, acc_sh, bacc_sh,
         isem, gsem, ssem, fgsem, fssem) = refs
    else:
        (h_hbm, eidx_hbm, zrow_hbm,
         out_hbm,
         idx_v, rows_v, wrow_v, acc_sh,
         isem, gsem, ssem) = refs

    c = lax.axis_index("c")
    s = lax.axis_index("s")

    for k in range(NBUF):
        pltpu.async_copy(eidx_hbm.at[s * NCHUNK + k], idx_v.at[k],
                         isem.at[k])

    pltpu.sync_copy(zrow_hbm, wrow_v)
    if with_bonds:
        pltpu.sync_copy(zbond_hbm, wbond_v)

    def zero_body(k, carry):
        r0 = s * WST + k * WCH
        pltpu.sync_copy(wrow_v, acc_sh.at[pl.ds(r0, WCH)])
        if with_bonds:
            pltpu.sync_copy(wbond_v, bacc_sh.at[pl.ds(r0, WCH)])
        return carry

    lax.fori_loop(0, NWO, zero_body, 0)

    @pl.when(s == NS - 1)
    def _():
        pltpu.sync_copy(wrow_v.at[pl.ds(0, TAIL)],
                        acc_sh.at[pl.ds(NS * WST, TAIL)])
        if with_bonds:
            pltpu.sync_copy(wbond_v.at[pl.ds(0, TAIL)],
                            bacc_sh.at[pl.ds(NS * WST, TAIL)])

    plsc.subcore_barrier()

    def wait_scatter(b):
        pltpu.make_async_copy(rows_v.at[b], acc_sh.at[idx_v.at[0, 1]],
                              ssem.at[b]).wait()

    def wait_idx(k):
        pltpu.make_async_copy(eidx_hbm.at[0], idx_v.at[k], isem.at[k]).wait()

    def wait_bscatter(q):
        pltpu.make_async_copy(fb_v.at[q], bacc_sh.at[idx_v.at[0, 1]],
                              fssem.at[q]).wait()

    def wait_gather(b):
        pltpu.make_async_copy(h_hbm.at[c].at[idx_v.at[0, 0]],
                              rows_v.at[b], gsem.at[b]).wait()

    wait_idx(0)
    pltpu.async_copy(h_hbm.at[c].at[idx_v.at[0, 0]], rows_v.at[0],
                     gsem.at[0])

    def group_body(g, carry):
        for bb in range(2 * NBUF):
            j = g * (2 * NBUF) + bb
            b = bb % NBUF
            k2 = (bb + NBUF) % (2 * NBUF)
            bn = (bb + 1) % NBUF
            kn = (bb + 1) % (2 * NBUF)
            jn = j + 1
            if with_bonds:
                my_par = (j % 2) == c

                @pl.when(jnp.logical_and(my_par, j >= NBUF))
                def _():
                    wait_bscatter(k2 // 2)

            @pl.when(jn < NCHUNK)
            def _():
                @pl.when(jn >= NBUF)
                def _():
                    wait_scatter(bn)
                wait_idx(kn)
                pltpu.async_copy(h_hbm.at[c].at[idx_v.at[kn, 0]],
                                 rows_v.at[bn], gsem.at[bn])

            @pl.when(j + NBUF < NCHUNK)
            def _():
                pltpu.async_copy(eidx_hbm.at[s * NCHUNK + j + NBUF],
                                 idx_v.at[k2], isem.at[k2])

            if with_bonds:
                @pl.when(my_par)
                def _():
                    pltpu.async_copy(fb_hbm.at[s * NCHUNK + j],
                                     fb_v.at[bb // 2], fgsem)
            wait_gather(b)
            pltpu.async_copy(rows_v.at[b], acc_sh.at[idx_v.at[bb, 1]],
                             ssem.at[b], add=True)
            if with_bonds:
                @pl.when(my_par)
                def _():
                    pltpu.make_async_copy(fb_hbm.at[0], fb_v.at[bb // 2],
                                          fgsem).wait()
                    pltpu.async_copy(fb_v.at[bb // 2],
                                     bacc_sh.at[idx_v.at[bb, 1]],
                                     fssem.at[bb // 2], add=True)
        return carry

    lax.fori_loop(0, NCHUNK // (2 * NBUF), group_body, 0)

    for b in range(NBUF):
        wait_scatter(b)
    if with_bonds:
        for q in (NBUF // 2, NBUF // 2 + 1):
            wait_bscatter(q)
    plsc.subcore_barrier()

    def wo_body(k, carry):
        r0 = s * WST + k * WCH
        pltpu.sync_copy(acc_sh.at[pl.ds(r0, WCH)], wrow_v)
        pltpu.sync_copy(wrow_v, out_hbm.at[c, pl.ds(r0, WCH)])
        if with_bonds:
            pltpu.sync_copy(bacc_sh.at[pl.ds(r0, WCH)], wbond_v)
            pltpu.sync_copy(wbond_v, bout_hbm.at[c, pl.ds(r0, WCH)])
        return carry

    lax.fori_loop(0, NWO, wo_body, 0)

    @pl.when(s == NS - 1)
    def _():
        r0 = NS * WST
        pltpu.sync_copy(acc_sh.at[pl.ds(r0, TAIL)], wrow_v.at[pl.ds(0, TAIL)])
        pltpu.sync_copy(wrow_v.at[pl.ds(0, TAIL)],
                        out_hbm.at[c, pl.ds(r0, TAIL)])
        if with_bonds:
            pltpu.sync_copy(bacc_sh.at[pl.ds(r0, TAIL)],
                            wbond_v.at[pl.ds(0, TAIL)])
            pltpu.sync_copy(wbond_v.at[pl.ds(0, TAIL)],
                            bout_hbm.at[c, pl.ds(r0, TAIL)])


def _make_seg_call(with_bonds):
    mesh = plsc.VectorSubcoreMesh(core_axis_name="c", subcore_axis_name="s",
                                  num_cores=NC, num_subcores=NS)
    common = [
        pltpu.VMEM((2 * NBUF, 2, CH), jnp.int32),
        pltpu.VMEM((NBUF, CH, HH), jnp.float32),
    ]
    if with_bonds:
        out_type = (jax.ShapeDtypeStruct((NC, N, HH), jnp.float32),
                    jax.ShapeDtypeStruct((NC, N, BOND), jnp.float32))
        scratch = common + [
            pltpu.VMEM((NBUF, CH, BOND), jnp.float32),
            pltpu.VMEM((WCH, HH), jnp.float32),
            pltpu.VMEM((WCH, BOND), jnp.float32),
            pltpu.VMEM_SHARED((N, HH), jnp.float32),
            pltpu.VMEM_SHARED((N, BOND), jnp.float32),
            pltpu.SemaphoreType.DMA((2 * NBUF,)),
            pltpu.SemaphoreType.DMA((NBUF,)),
            pltpu.SemaphoreType.DMA((NBUF,)),
            pltpu.SemaphoreType.DMA,
            pltpu.SemaphoreType.DMA((NBUF,)),
        ]
    else:
        out_type = jax.ShapeDtypeStruct((NC, N, HH), jnp.float32)
        scratch = common + [
            pltpu.VMEM((WCH, HH), jnp.float32),
            pltpu.VMEM_SHARED((N, HH), jnp.float32),
            pltpu.SemaphoreType.DMA((2 * NBUF,)),
            pltpu.SemaphoreType.DMA((NBUF,)),
            pltpu.SemaphoreType.DMA((NBUF,)),
        ]
    return pl.kernel(functools.partial(_seg_body, with_bonds),
                     out_type=out_type, mesh=mesh, scratch_types=scratch,
                     compiler_params=pltpu.CompilerParams(
                         use_tc_tiling_on_sc=False))


@functools.lru_cache(maxsize=None)
def _seg_call(with_bonds):
    return _make_seg_call(with_bonds)


_ROW_SPEC = pl.BlockSpec((BN, H), lambda i: (i, 0))
_PAIR_SPEC = pl.BlockSpec((NC, BN, HH), lambda i: (0, i, 0))


def _split_store(o_ref, x):
    o_ref[0] = x[:, :HH]
    o_ref[1] = x[:, HH:]


_EB = E // (N // BN)
_CB = _EB // CH


def _h0_kernel(fa_ref, wi_ref, fbt_ref, o_ref, fb3_ref):
    _split_store(o_ref, jax.nn.relu(
        jnp.dot(fa_ref[...], wi_ref[...], preferred_element_type=jnp.float32)))
    fb3_ref[...] = fbt_ref[...].T.reshape(_CB, CH, BOND)


def _h0_call(f_atoms, W_i, fbT):
    return pl.pallas_call(
        _h0_kernel,
        grid=(N // BN,),
        in_specs=[pl.BlockSpec((BN, ATOM), lambda i: (i, 0)),
                  pl.BlockSpec((ATOM, H), lambda i: (0, 0)),
                  pl.BlockSpec((BOND, _EB), lambda i: (0, i))],
        out_specs=[_PAIR_SPEC,
                   pl.BlockSpec((_CB, CH, BOND), lambda i: (i, 0, 0))],
        out_shape=[jax.ShapeDtypeStruct((NC, N, HH), jnp.float32),
                   jax.ShapeDtypeStruct((E // CH, CH, BOND), jnp.float32)],
    )(f_atoms, W_i, fbT)


def _pair_cat(ref):
    return jnp.concatenate([ref[0], ref[1]], axis=1)


def _upd1_kernel(h0_ref, m_ref, bs_ref, wh1_ref, wh2_ref, h_ref, bt_ref):
    bt = jnp.dot(bs_ref[0] + bs_ref[1], wh2_ref[...],
                 preferred_element_type=jnp.float32)
    acc = jnp.dot(_pair_cat(m_ref), wh1_ref[...],
                  preferred_element_type=jnp.float32)
    bt_ref[...] = bt
    _split_store(h_ref, jax.nn.relu(_pair_cat(h0_ref) + acc + bt))


def _upd1_call(h0, m, bsum, Wh1, Wh2):
    return pl.pallas_call(
        _upd1_kernel,
        grid=(N // BN,),
        in_specs=[_PAIR_SPEC,
                  _PAIR_SPEC,
                  pl.BlockSpec((NC, BN, BOND), lambda i: (0, i, 0)),
                  pl.BlockSpec((H, H), lambda i: (0, 0)),
                  pl.BlockSpec((BOND, H), lambda i: (0, 0))],
        out_specs=[_PAIR_SPEC, _ROW_SPEC],
        out_shape=[jax.ShapeDtypeStruct((NC, N, HH), jnp.float32),
                   jax.ShapeDtypeStruct((N, H), jnp.float32)],
    )(h0, m, bsum, Wh1, Wh2)


def _upd2_kernel(h0_ref, m_ref, bt_ref, wh1_ref, h_ref):
    acc = jnp.dot(_pair_cat(m_ref), wh1_ref[...],
                  preferred_element_type=jnp.float32)
    _split_store(h_ref, jax.nn.relu(_pair_cat(h0_ref) + acc + bt_ref[...]))


def _upd2_call(h0, m, bt, Wh1):
    return pl.pallas_call(
        _upd2_kernel,
        grid=(N // BN,),
        in_specs=[_PAIR_SPEC,
                  _PAIR_SPEC,
                  _ROW_SPEC,
                  pl.BlockSpec((H, H), lambda i: (0, 0))],
        out_specs=_PAIR_SPEC,
        out_shape=jax.ShapeDtypeStruct((NC, N, HH), jnp.float32),
    )(h0, m, bt, Wh1)


def _final_kernel(fa_ref, m_ref, mol_ref, wo1_ref, wo2_ref,
                  w1_ref, b1_ref, w2_ref, b2_ref, o_ref,
                  sums_ref, counts_ref):
    i = pl.program_id(0)

    @pl.when(i == 0)
    def _():
        sums_ref[...] = jnp.zeros_like(sums_ref)
        counts_ref[...] = jnp.zeros_like(counts_ref)

    a = jax.nn.relu(
        jnp.dot(fa_ref[...], wo1_ref[...], preferred_element_type=jnp.float32)
        + jnp.dot(_pair_cat(m_ref), wo2_ref[...],
                  preferred_element_type=jnp.float32))
    ids = mol_ref[...]
    onehot = (lax.broadcasted_iota(jnp.int32, (BN, M), 1) == ids
              ).astype(jnp.float32)
    sums_ref[...] += lax.dot_general(
        onehot, a, (((0,), (0,)), ((), ())),
        preferred_element_type=jnp.float32,
        precision=lax.Precision.HIGHEST)
    counts_ref[...] += lax.dot_general(
        onehot, jnp.ones((BN, H), jnp.float32), (((0,), (0,)), ((), ())),
        preferred_element_type=jnp.float32,
        precision=lax.Precision.HIGHEST)

    @pl.when(i == pl.num_programs(0) - 1)
    def _():
        mol_vecs = sums_ref[...] / jnp.maximum(counts_ref[...], 1.0)
        hidden = jax.nn.relu(
            jnp.dot(mol_vecs, w1_ref[...], preferred_element_type=jnp.float32)
            + b1_ref[...])
        o_ref[...] = (jnp.dot(hidden, w2_ref[...],
                              preferred_element_type=jnp.float32)
                      + b2_ref[...])


def _final_call(f_atoms, m, mol, Wo1, Wo2, W1, b1, W2, b2):
    return pl.pallas_call(
        _final_kernel,
        grid=(N // BN,),
        in_specs=[pl.BlockSpec((BN, ATOM), lambda i: (i, 0)),
                  _PAIR_SPEC,
                  pl.BlockSpec((BN, 1), lambda i: (i, 0)),
                  pl.BlockSpec((ATOM, H), lambda i: (0, 0)),
                  pl.BlockSpec((H, H), lambda i: (0, 0)),
                  pl.BlockSpec((H, FFN), lambda i: (0, 0)),
                  pl.BlockSpec((1, FFN), lambda i: (0, 0)),
                  pl.BlockSpec((FFN, 1), lambda i: (0, 0)),
                  pl.BlockSpec((1, 1), lambda i: (0, 0))],
        out_specs=pl.BlockSpec((M, 1), lambda i: (0, 0)),
        out_shape=jax.ShapeDtypeStruct((M, 1), jnp.float32),
        scratch_shapes=[pltpu.VMEM((M, H), jnp.float32),
                        pltpu.VMEM((M, H), jnp.float32)],
    )(f_atoms, m, mol, Wo1, Wo2, W1, b1, W2, b2)


def kernel(f_atoms, f_bonds, edge_index, mol_ids, W_i, W_h, W_o,
           ffn_W1, ffn_b1, ffn_W2, ffn_b2):
    src = edge_index[0].astype(jnp.int32)
    dst = edge_index[1].astype(jnp.int32)
    eidx = jnp.stack([src.reshape(E // CH, CH), dst.reshape(E // CH, CH)],
                     axis=1)
    mol = mol_ids.astype(jnp.int32).reshape(N, 1)
    zrow = jnp.zeros((WCH, HH), jnp.float32)
    zbond = jnp.zeros((WCH, BOND), jnp.float32)
    Wh1, Wh2 = W_h[:H], W_h[H:]
    Wo1, Wo2 = W_o[:ATOM], W_o[ATOM:]

    h0, fb3 = _h0_call(f_atoms, W_i, f_bonds.T)
    m, bsum = _seg_call(True)(h0, eidx, fb3, zrow, zbond)
    h1, bt = _upd1_call(h0, m, bsum, Wh1, Wh2)
    m2 = _seg_call(False)(h1, eidx, zrow)
    h2 = _upd2_call(h0, m2, bt, Wh1)
    m3 = _seg_call(False)(h2, eidx, zrow)
    return _final_call(f_atoms, m3, mol, Wo1, Wo2,
                       ffn_W1, ffn_b1.reshape(1, FFN),
                       ffn_W2, ffn_b2.reshape(1, 1))


# speedup vs baseline: 1.0828x; 1.0828x over previous
import functools

import jax
import jax.numpy as jnp
from jax import lax
from jax.experimental import pallas as pl
from jax.experimental.pallas import tpu as pltpu
from jax.experimental.pallas import tpu_sc as plsc

H = 128
HH = H // 2
BOND = 16
ATOM = 128
N = 10000
E = 320000
M = 256
FFN = 300
NC, NS = 2, 16
E_S = E // NS
CH = 125
NCHUNK = E_S // CH
NBUF = 4
WST = 624
WCH = 104
NWO = WST // WCH
TAIL = N - NS * WST
BN = 1000


def _seg_body(with_bonds, *refs):
    if with_bonds:
        (h_hbm, eidx_hbm, fb_hbm, zrow_hbm, zbond_hbm,
         out_hbm, bout_hbm,
         idx_v, rows_v, fb_v, wrow_v, wbond_v, acc_sh, bacc_sh,
         isem, gsem, ssem, fgsem, fssem) = refs
    else:
        (h_hbm, eidx_hbm, zrow_hbm,
         out_hbm,
         idx_v, rows_v, wrow_v, acc_sh,
         isem, gsem, ssem) = refs

    c = lax.axis_index("c")
    s = lax.axis_index("s")

    for k in range(NBUF):
        pltpu.async_copy(eidx_hbm.at[s * NCHUNK + k], idx_v.at[k],
                         isem.at[k])

    pltpu.sync_copy(zrow_hbm, wrow_v)
    if with_bonds:
        pltpu.sync_copy(zbond_hbm, wbond_v)

    def zero_body(k, carry):
        r0 = s * WST + k * WCH
        pltpu.sync_copy(wrow_v, acc_sh.at[pl.ds(r0, WCH)])
        if with_bonds:
            pltpu.sync_copy(wbond_v, bacc_sh.at[pl.ds(r0, WCH)])
        return carry

    lax.fori_loop(0, NWO, zero_body, 0)

    @pl.when(s == NS - 1)
    def _():
        pltpu.sync_copy(wrow_v.at[pl.ds(0, TAIL)],
                        acc_sh.at[pl.ds(NS * WST, TAIL)])
        if with_bonds:
            pltpu.sync_copy(wbond_v.at[pl.ds(0, TAIL)],
                            bacc_sh.at[pl.ds(NS * WST, TAIL)])

    plsc.subcore_barrier()

    def wait_scatter(b):
        pltpu.make_async_copy(rows_v.at[b], acc_sh.at[idx_v.at[0, 1]],
                              ssem.at[b]).wait()

    def wait_idx(k):
        pltpu.make_async_copy(eidx_hbm.at[0], idx_v.at[k], isem.at[k]).wait()

    def wait_bscatter(q):
        pltpu.make_async_copy(fb_v.at[q], bacc_sh.at[idx_v.at[0, 1]],
                              fssem.at[q]).wait()

    def wait_gather(b):
        pltpu.make_async_copy(h_hbm.at[c].at[idx_v.at[0, 0]],
                              rows_v.at[b], gsem.at[b]).wait()

    wait_idx(0)
    pltpu.async_copy(h_hbm.at[c].at[idx_v.at[0, 0]], rows_v.at[0],
                     gsem.at[0])

    def group_body(g, carry):
        for bb in range(2 * NBUF):
            j = g * (2 * NBUF) + bb
            b = bb % NBUF
            k2 = (bb + NBUF) % (2 * NBUF)
            bn = (bb + 1) % NBUF
            kn = (bb + 1) % (2 * NBUF)
            jn = j + 1
            if with_bonds:
                my_par = (j % 2) == c

                @pl.when(jnp.logical_and(my_par, j >= NBUF))
                def _():
                    wait_bscatter(k2 // 2)

            @pl.when(jn < NCHUNK)
            def _():
                @pl.when(jn >= NBUF)
                def _():
                    wait_scatter(bn)
                wait_idx(kn)
                pltpu.async_copy(h_hbm.at[c].at[idx_v.at[kn, 0]],
                                 rows_v.at[bn], gsem.at[bn])

            @pl.when(j + NBUF < NCHUNK)
            def _():
                pltpu.async_copy(eidx_hbm.at[s * NCHUNK + j + NBUF],
                                 idx_v.at[k2], isem.at[k2])

            if with_bonds:
                @pl.when(my_par)
                def _():
                    pltpu.async_copy(fb_hbm.at[s * NCHUNK + j],
                                     fb_v.at[bb // 2], fgsem)
            wait_gather(b)
            pltpu.async_copy(rows_v.at[b], acc_sh.at[idx_v.at[bb, 1]],
                             ssem.at[b], add=True)
            if with_bonds:
                @pl.when(my_par)
                def _():
                    pltpu.make_async_copy(fb_hbm.at[0], fb_v.at[bb // 2],
                                          fgsem).wait()
                    pltpu.async_copy(fb_v.at[bb // 2],
                                     bacc_sh.at[idx_v.at[bb, 1]],
                                     fssem.at[bb // 2], add=True)
        return carry

    lax.fori_loop(0, NCHUNK // (2 * NBUF), group_body, 0)

    for b in range(NBUF):
        wait_scatter(b)
    if with_bonds:
        for q in (NBUF // 2, NBUF // 2 + 1):
            wait_bscatter(q)
    plsc.subcore_barrier()

    def wo_body(k, carry):
        r0 = s * WST + k * WCH
        pltpu.sync_copy(acc_sh.at[pl.ds(r0, WCH)], wrow_v)
        pltpu.sync_copy(wrow_v, out_hbm.at[c, pl.ds(r0, WCH)])
        if with_bonds:
            pltpu.sync_copy(bacc_sh.at[pl.ds(r0, WCH)], wbond_v)
            pltpu.sync_copy(wbond_v, bout_hbm.at[c, pl.ds(r0, WCH)])
        return carry

    lax.fori_loop(0, NWO, wo_body, 0)

    @pl.when(s == NS - 1)
    def _():
        r0 = NS * WST
        pltpu.sync_copy(acc_sh.at[pl.ds(r0, TAIL)], wrow_v.at[pl.ds(0, TAIL)])
        pltpu.sync_copy(wrow_v.at[pl.ds(0, TAIL)],
                        out_hbm.at[c, pl.ds(r0, TAIL)])
        if with_bonds:
            pltpu.sync_copy(bacc_sh.at[pl.ds(r0, TAIL)],
                            wbond_v.at[pl.ds(0, TAIL)])
            pltpu.sync_copy(wbond_v.at[pl.ds(0, TAIL)],
                            bout_hbm.at[c, pl.ds(r0, TAIL)])


def _make_seg_call(with_bonds):
    mesh = plsc.VectorSubcoreMesh(core_axis_name="c", subcore_axis_name="s",
                                  num_cores=NC, num_subcores=NS)
    common = [
        pltpu.VMEM((2 * NBUF, 2, CH), jnp.int32),
        pltpu.VMEM((NBUF, CH, HH), jnp.float32),
    ]
    if with_bonds:
        out_type = (jax.ShapeDtypeStruct((NC, N, HH), jnp.float32),
                    jax.ShapeDtypeStruct((NC, N, BOND), jnp.float32))
        scratch = common + [
            pltpu.VMEM((NBUF, CH, BOND), jnp.float32),
            pltpu.VMEM((WCH, HH), jnp.float32),
            pltpu.VMEM((WCH, BOND), jnp.float32),
            pltpu.VMEM_SHARED((N, HH), jnp.float32),
            pltpu.VMEM_SHARED((N, BOND), jnp.float32),
            pltpu.SemaphoreType.DMA((2 * NBUF,)),
            pltpu.SemaphoreType.DMA((NBUF,)),
            pltpu.SemaphoreType.DMA((NBUF,)),
            pltpu.SemaphoreType.DMA,
            pltpu.SemaphoreType.DMA((NBUF,)),
        ]
    else:
        out_type = jax.ShapeDtypeStruct((NC, N, HH), jnp.float32)
        scratch = common + [
            pltpu.VMEM((WCH, HH), jnp.float32),
            pltpu.VMEM_SHARED((N, HH), jnp.float32),
            pltpu.SemaphoreType.DMA((2 * NBUF,)),
            pltpu.SemaphoreType.DMA((NBUF,)),
            pltpu.SemaphoreType.DMA((NBUF,)),
        ]
    return pl.kernel(functools.partial(_seg_body, with_bonds),
                     out_type=out_type, mesh=mesh, scratch_types=scratch,
                     compiler_params=pltpu.CompilerParams(
                         use_tc_tiling_on_sc=False))


@functools.lru_cache(maxsize=None)
def _seg_call(with_bonds):
    return _make_seg_call(with_bonds)


_ROW_SPEC = pl.BlockSpec((BN, H), lambda i: (i, 0))
_PAIR_SPEC = pl.BlockSpec((NC, BN, HH), lambda i: (0, i, 0))


def _split_store(o_ref, x):
    o_ref[0] = x[:, :HH]
    o_ref[1] = x[:, HH:]


def _h0_kernel(fa_ref, wi_ref, o_ref):
    _split_store(o_ref, jax.nn.relu(
        jnp.dot(fa_ref[...], wi_ref[...], preferred_element_type=jnp.float32)))


def _h0_call(f_atoms, W_i):
    return pl.pallas_call(
        _h0_kernel,
        grid=(N // BN,),
        in_specs=[pl.BlockSpec((BN, ATOM), lambda i: (i, 0)),
                  pl.BlockSpec((ATOM, H), lambda i: (0, 0))],
        out_specs=_PAIR_SPEC,
        out_shape=jax.ShapeDtypeStruct((NC, N, HH), jnp.float32),
    )(f_atoms, W_i)


def _pair_cat(ref):
    return jnp.concatenate([ref[0], ref[1]], axis=1)


def _upd1_kernel(h0_ref, m_ref, bs_ref, wh1_ref, wh2_ref, h_ref, bt_ref):
    bt = jnp.dot(bs_ref[0] + bs_ref[1], wh2_ref[...],
                 preferred_element_type=jnp.float32)
    acc = jnp.dot(_pair_cat(m_ref), wh1_ref[...],
                  preferred_element_type=jnp.float32)
    bt_ref[...] = bt
    _split_store(h_ref, jax.nn.relu(_pair_cat(h0_ref) + acc + bt))


def _upd1_call(h0, m, bsum, Wh1, Wh2):
    return pl.pallas_call(
        _upd1_kernel,
        grid=(N // BN,),
        in_specs=[_PAIR_SPEC,
                  _PAIR_SPEC,
                  pl.BlockSpec((NC, BN, BOND), lambda i: (0, i, 0)),
                  pl.BlockSpec((H, H), lambda i: (0, 0)),
                  pl.BlockSpec((BOND, H), lambda i: (0, 0))],
        out_specs=[_PAIR_SPEC, _ROW_SPEC],
        out_shape=[jax.ShapeDtypeStruct((NC, N, HH), jnp.float32),
                   jax.ShapeDtypeStruct((N, H), jnp.float32)],
    )(h0, m, bsum, Wh1, Wh2)


def _upd2_kernel(h0_ref, m_ref, bt_ref, wh1_ref, h_ref):
    acc = jnp.dot(_pair_cat(m_ref), wh1_ref[...],
                  preferred_element_type=jnp.float32)
    _split_store(h_ref, jax.nn.relu(_pair_cat(h0_ref) + acc + bt_ref[...]))


def _upd2_call(h0, m, bt, Wh1):
    return pl.pallas_call(
        _upd2_kernel,
        grid=(N // BN,),
        in_specs=[_PAIR_SPEC,
                  _PAIR_SPEC,
                  _ROW_SPEC,
                  pl.BlockSpec((H, H), lambda i: (0, 0))],
        out_specs=_PAIR_SPEC,
        out_shape=jax.ShapeDtypeStruct((NC, N, HH), jnp.float32),
    )(h0, m, bt, Wh1)


def _final_kernel(fa_ref, m_ref, mol_ref, wo1_ref, wo2_ref,
                  w1_ref, b1_ref, w2_ref, b2_ref, o_ref,
                  sums_ref, counts_ref):
    i = pl.program_id(0)

    @pl.when(i == 0)
    def _():
        sums_ref[...] = jnp.zeros_like(sums_ref)
        counts_ref[...] = jnp.zeros_like(counts_ref)

    a = jax.nn.relu(
        jnp.dot(fa_ref[...], wo1_ref[...], preferred_element_type=jnp.float32)
        + jnp.dot(_pair_cat(m_ref), wo2_ref[...],
                  preferred_element_type=jnp.float32))
    ids = mol_ref[...]
    onehot = (lax.broadcasted_iota(jnp.int32, (BN, M), 1) == ids
              ).astype(jnp.float32)
    sums_ref[...] += lax.dot_general(
        onehot, a, (((0,), (0,)), ((), ())),
        preferred_element_type=jnp.float32,
        precision=lax.Precision.HIGHEST)
    counts_ref[...] += lax.dot_general(
        onehot, jnp.ones((BN, H), jnp.float32), (((0,), (0,)), ((), ())),
        preferred_element_type=jnp.float32,
        precision=lax.Precision.HIGHEST)

    @pl.when(i == pl.num_programs(0) - 1)
    def _():
        mol_vecs = sums_ref[...] / jnp.maximum(counts_ref[...], 1.0)
        hidden = jax.nn.relu(
            jnp.dot(mol_vecs, w1_ref[...], preferred_element_type=jnp.float32)
            + b1_ref[...])
        o_ref[...] = (jnp.dot(hidden, w2_ref[...],
                              preferred_element_type=jnp.float32)
                      + b2_ref[...])


def _final_call(f_atoms, m, mol, Wo1, Wo2, W1, b1, W2, b2):
    return pl.pallas_call(
        _final_kernel,
        grid=(N // BN,),
        in_specs=[pl.BlockSpec((BN, ATOM), lambda i: (i, 0)),
                  _PAIR_SPEC,
                  pl.BlockSpec((BN, 1), lambda i: (i, 0)),
                  pl.BlockSpec((ATOM, H), lambda i: (0, 0)),
                  pl.BlockSpec((H, H), lambda i: (0, 0)),
                  pl.BlockSpec((H, FFN), lambda i: (0, 0)),
                  pl.BlockSpec((1, FFN), lambda i: (0, 0)),
                  pl.BlockSpec((FFN, 1), lambda i: (0, 0)),
                  pl.BlockSpec((1, 1), lambda i: (0, 0))],
        out_specs=pl.BlockSpec((M, 1), lambda i: (0, 0)),
        out_shape=jax.ShapeDtypeStruct((M, 1), jnp.float32),
        scratch_shapes=[pltpu.VMEM((M, H), jnp.float32),
                        pltpu.VMEM((M, H), jnp.float32)],
    )(f_atoms, m, mol, Wo1, Wo2, W1, b1, W2, b2)


def kernel(f_atoms, f_bonds, edge_index, mol_ids, W_i, W_h, W_o,
           ffn_W1, ffn_b1, ffn_W2, ffn_b2):
    src = edge_index[0].astype(jnp.int32)
    dst = edge_index[1].astype(jnp.int32)
    eidx = jnp.stack([src.reshape(E // CH, CH), dst.reshape(E // CH, CH)],
                     axis=1)
    fb3 = f_bonds.reshape(E // CH, CH, BOND)
    mol = mol_ids.astype(jnp.int32).reshape(N, 1)
    zrow = jnp.zeros((WCH, HH), jnp.float32)
    zbond = jnp.zeros((WCH, BOND), jnp.float32)
    Wh1, Wh2 = W_h[:H], W_h[H:]
    Wo1, Wo2 = W_o[:ATOM], W_o[ATOM:]

    h0 = _h0_call(f_atoms, W_i)
    m, bsum = _seg_call(True)(h0, eidx, fb3, zrow, zbond)
    h1, bt = _upd1_call(h0, m, bsum, Wh1, Wh2)
    m2 = _seg_call(False)(h1, eidx, zrow)
    h2 = _upd2_call(h0, m2, bt, Wh1)
    m3 = _seg_call(False)(h2, eidx, zrow)
    return _final_call(f_atoms, m3, mol, Wo1, Wo2,
                       ffn_W1, ffn_b1.reshape(1, FFN),
                       ffn_W2, ffn_b2.reshape(1, 1))
